# baseline (speedup 1.0000x reference)
import functools

import jax
import jax.numpy as jnp
from jax import lax
from jax.experimental import pallas as pl
from jax.experimental.pallas import tpu as pltpu
from jax.experimental.pallas import tpu_sc as plsc

CAT_COLS = (1, 3, 4, 6, 7)
NUM_COLS = (0, 2, 5, 8, 9, 10, 11, 12, 13)
CAT_OFF = (1, 7, 12, 18, 23)
NUM_OFF = (0, 6, 17, 28, 29, 30, 31, 32, 33)
NUM_FEAT = 14
VOCAB = 1_000_000
B, L = 16384, 50
E = B * L
NW = 32
PER_W = E // NW
CHUNK = 3200
NCHUNK = PER_W // CHUNK
GRP = CHUNK // 16
NROW = CHUNK // 128

EMB_DIM = 5
SCAL_COLS = 1280
SCAL_ROWS = 3912
SCAL_BLKR = 256


def _scalarize_body(e1, e2, e3, e4, e5, m1, m2, m3, m4, m5, s1, s2, s3, s4, s5):
    mrefs = (m1, m2, m3, m4, m5)
    srefs = (s1, s2, s3, s4, s5)
    for t, eref in enumerate((e1, e2, e3, e4, e5)):
        srefs[t][...] = jnp.dot(eref[...], mrefs[t][...],
                                preferred_element_type=jnp.float32)


def _scalarize(embs, W):
    pad = SCAL_ROWS * SCAL_COLS - VOCAB * 5
    efs = [jnp.concatenate([e.reshape(-1), jnp.zeros((pad,), jnp.float32)])
           .astype(jnp.bfloat16).reshape(SCAL_ROWS, SCAL_COLS) for e in embs]
    jj = jnp.arange(SCAL_COLS)[:, None]
    kk = jnp.arange(256)[None, :]
    d = jj - 5 * kk
    msels = []
    for off in CAT_OFF:
        m = jnp.zeros((SCAL_COLS, 256), jnp.float32)
        for di in range(EMB_DIM):
            m = m + jnp.where(d == di, W[off + di, 0], 0.0)
        msels.append(m.astype(jnp.bfloat16))
    eblk = pl.BlockSpec((SCAL_BLKR, SCAL_COLS), lambda i: (i, 0))
    mblk = pl.BlockSpec((SCAL_COLS, 256), lambda i: (0, 0))
    sblk = pl.BlockSpec((SCAL_BLKR, 256), lambda i: (i, 0))
    s2d = pl.pallas_call(
        _scalarize_body,
        grid=(pl.cdiv(SCAL_ROWS, SCAL_BLKR),),
        in_specs=[eblk] * 5 + [mblk] * 5,
        out_specs=[sblk] * 5,
        out_shape=[jax.ShapeDtypeStruct((SCAL_ROWS, 256), jnp.float32)] * 5,
    )(*efs, *msels)
    return [s.reshape(-1) for s in s2d]


def _sc_body(xf, s1, s2, s3, s4, s5, wb, out, xv, accv, i1, i2, i3, i4, i5,
             g1, g2, g3, g4, g5, outv, wv, sem):
    srefs = (s1, s2, s3, s4, s5)
    irefs = (i1, i2, i3, i4, i5)
    grefs = (g1, g2, g3, g4, g5)
    wid = lax.axis_index("s") * 2 + lax.axis_index("c")
    iota = lax.iota(jnp.int32, 16)

    def bf16_round(v):
        u = plsc.bitcast(v, jnp.uint32)
        u = (u + jnp.uint32(0x7FFF) + ((u >> jnp.uint32(16)) & jnp.uint32(1)))
        u = u & jnp.uint32(0xFFFF0000)
        return plsc.bitcast(u, jnp.float32)
    pltpu.sync_copy(wb, wv)
    w_all = wv[...]
    wsp = [jnp.broadcast_to(w_all[j], (16,)) for j in range(9)]
    bsp = jnp.broadcast_to(w_all[9], (16,))
    piota = iota * NUM_FEAT

    def chunk_body(ck, _):
        base_e = wid * PER_W + ck * CHUNK
        pltpu.sync_copy(xf.at[pl.ds(base_e * NUM_FEAT, CHUNK * NUM_FEAT)], xv)

        def grp_body(j, _):
            for k in range(8):
                pos = piota + (j * 128 + k * 16) * NUM_FEAT
                acc = bsp
                for t in range(9):
                    v = plsc.load_gather(xv, [pos + NUM_COLS[t]])
                    acc = acc + bf16_round(v) * wsp[t]
                for t in range(5):
                    v = plsc.load_gather(xv, [pos + CAT_COLS[t]])
                    irefs[t][j, pl.ds(k * 16, 16)] = v.astype(jnp.int32)
                accv[pl.ds(j * 128 + k * 16, 16)] = acc
            return 0

        lax.fori_loop(0, NROW, grp_body, 0)

        def fire_body(j, _):
            for t in range(5):
                pltpu.make_async_copy(
                    srefs[t].at[irefs[t].at[j]],
                    grefs[t].at[j], sem).start()
            for t in range(5):
                pltpu.make_async_copy(
                    srefs[t].at[irefs[t].at[j]],
                    grefs[t].at[j], sem).wait()
            return 0

        lax.fori_loop(0, NROW, fire_body, 0)

        def out_body(j, _):
            for k in range(8):
                sl = pl.ds(k * 16, 16)
                z = (accv[pl.ds(j * 128 + k * 16, 16)] + g1[j, sl] + g2[j, sl]
                     + g3[j, sl] + g4[j, sl] + g5[j, sl])
                z = jnp.minimum(jnp.maximum(z, -30.0), 30.0)
                outv[pl.ds(j * 128 + k * 16, 16)] = 1.0 / (1.0 + jnp.exp(-z))
            return 0

        lax.fori_loop(0, NROW, out_body, 0)
        pltpu.sync_copy(outv, out.at[pl.ds(base_e, CHUNK)])
        return 0

    lax.fori_loop(0, NCHUNK, chunk_body, 0)


def _sc_gather(xf, svals, wb):
    mesh = plsc.VectorSubcoreMesh(core_axis_name="c", subcore_axis_name="s")
    f = functools.partial(
        pl.kernel, _sc_body, mesh=mesh,
        out_type=jax.ShapeDtypeStruct((E,), jnp.float32),
        scratch_types=[
            pltpu.VMEM((CHUNK * NUM_FEAT,), jnp.float32),
            pltpu.VMEM((CHUNK,), jnp.float32),
        ] + [pltpu.VMEM((NROW, 128), jnp.int32)] * 5
          + [pltpu.VMEM((NROW, 128), jnp.float32)] * 5
          + [pltpu.VMEM((CHUNK,), jnp.float32),
             pltpu.VMEM((16,), jnp.float32),
             pltpu.SemaphoreType.DMA],
        compiler_params=pltpu.CompilerParams(needs_layout_passes=False),
    )()
    return f(xf, *svals, wb)


def kernel(x, emb_1, emb_3, emb_4, emb_6, emb_7, W, b):
    svals = _scalarize((emb_1, emb_3, emb_4, emb_6, emb_7), W)
    wnum = jnp.stack([W[o, 0] for o in NUM_OFF])
    wnum = wnum.astype(jnp.bfloat16).astype(jnp.float32)
    wb = jnp.concatenate([wnum, b, jnp.zeros((6,), jnp.float32)])
    xf = x.reshape(-1)
    o = _sc_gather(xf, svals, wb)
    return o.reshape(B, L)


# speedup vs baseline: 2.8510x; 2.8510x over previous
import functools

import jax
import jax.numpy as jnp
from jax import lax
from jax.experimental import pallas as pl
from jax.experimental.pallas import tpu as pltpu
from jax.experimental.pallas import tpu_sc as plsc

CAT_COLS = (1, 3, 4, 6, 7)
NUM_COLS = (0, 2, 5, 8, 9, 10, 11, 12, 13)
CAT_OFF = (1, 7, 12, 18, 23)
NUM_OFF = (0, 6, 17, 28, 29, 30, 31, 32, 33)
NUM_FEAT = 14
VOCAB = 1_000_000
B, L = 16384, 50
E = B * L
NW = 32
PER_W = E // NW
CHUNK = 1280
NCHUNK = PER_W // CHUNK
NROW = CHUNK // 128

EMB_DIM = 5
SCAL_COLS = 1280
SCAL_ROWS = 3912
SCAL_BLKR = 256


def _scalarize_body(e1, e2, e3, e4, e5, m1, m2, m3, m4, m5, s1, s2, s3, s4, s5):
    mrefs = (m1, m2, m3, m4, m5)
    srefs = (s1, s2, s3, s4, s5)
    for t, eref in enumerate((e1, e2, e3, e4, e5)):
        srefs[t][...] = jnp.dot(eref[...], mrefs[t][...],
                                preferred_element_type=jnp.float32)


def _scalarize(embs, W):
    pad = SCAL_ROWS * SCAL_COLS - VOCAB * 5
    efs = [jnp.concatenate([e.reshape(-1), jnp.zeros((pad,), jnp.float32)])
           .reshape(SCAL_ROWS, SCAL_COLS) for e in embs]
    jj = jnp.arange(SCAL_COLS)[:, None]
    kk = jnp.arange(256)[None, :]
    d = jj - 5 * kk
    msels = []
    for off in CAT_OFF:
        m = jnp.zeros((SCAL_COLS, 256), jnp.float32)
        for di in range(EMB_DIM):
            m = m + jnp.where(d == di, W[off + di, 0], 0.0)
        msels.append(m)
    eblk = pl.BlockSpec((SCAL_BLKR, SCAL_COLS), lambda i: (i, 0))
    mblk = pl.BlockSpec((SCAL_COLS, 256), lambda i: (0, 0))
    sblk = pl.BlockSpec((SCAL_BLKR, 256), lambda i: (i, 0))
    s2d = pl.pallas_call(
        _scalarize_body,
        grid=(pl.cdiv(SCAL_ROWS, SCAL_BLKR),),
        in_specs=[eblk] * 5 + [mblk] * 5,
        out_specs=[sblk] * 5,
        out_shape=[jax.ShapeDtypeStruct((SCAL_ROWS, 256), jnp.float32)] * 5,
    )(*efs, *msels)
    return [s.reshape(-1) for s in s2d]


def _sc_body(xf, s1, s2, s3, s4, s5, wb, out, xv, accv, i1, i2, i3, i4, i5,
             g1, g2, g3, g4, g5, outv, wv, sem):
    srefs = (s1, s2, s3, s4, s5)
    irefs = (i1, i2, i3, i4, i5)
    grefs = (g1, g2, g3, g4, g5)
    wid = lax.axis_index("s") * 2 + lax.axis_index("c")
    iota = lax.iota(jnp.int32, 16)

    def bf16_round(v):
        u = plsc.bitcast(v, jnp.uint32)
        u = (u + jnp.uint32(0x7FFF) + ((u >> jnp.uint32(16)) & jnp.uint32(1)))
        u = u & jnp.uint32(0xFFFF0000)
        return plsc.bitcast(u, jnp.float32)
    pltpu.sync_copy(wb, wv)
    w_all = wv[...]
    wsp = [jnp.broadcast_to(w_all[j], (16,)) for j in range(9)]
    bsp = jnp.broadcast_to(w_all[9], (16,))
    piota = iota * NUM_FEAT

    def extract(p, base_e):
        xoff = p * CHUNK * NUM_FEAT
        pltpu.sync_copy(xf.at[pl.ds(base_e * NUM_FEAT, CHUNK * NUM_FEAT)],
                        xv.at[pl.ds(xoff, CHUNK * NUM_FEAT)])

        def grp_body(j, _):
            for k in range(8):
                pos = piota + xoff + (j * 128 + k * 16) * NUM_FEAT
                acc = bsp
                for t in range(9):
                    v = plsc.load_gather(xv, [pos + NUM_COLS[t]])
                    acc = acc + bf16_round(v) * wsp[t]
                for t in range(5):
                    v = plsc.load_gather(xv, [pos + CAT_COLS[t]])
                    irefs[t][p * NROW + j, pl.ds(k * 16, 16)] = v.astype(jnp.int32)
                accv[pl.ds(p * CHUNK + j * 128 + k * 16, 16)] = acc
            return 0

        lax.fori_loop(0, NROW, grp_body, 0)

    def fire(p):
        def fire_body(j, _):
            for t in range(5):
                pltpu.make_async_copy(srefs[t].at[irefs[t].at[p * NROW + j]],
                                      grefs[t].at[p * NROW + j], sem).start()
            return 0

        lax.fori_loop(0, NROW, fire_body, 0)

    def drain(p):
        def drain_body(j, _):
            for t in range(5):
                pltpu.make_async_copy(srefs[t].at[irefs[t].at[p * NROW + j]],
                                      grefs[t].at[p * NROW + j], sem).wait()
            return 0

        lax.fori_loop(0, NROW, drain_body, 0)

    def combine(p, base_e):
        def out_body(j, _):
            r = p * NROW + j
            for k in range(8):
                sl = pl.ds(k * 16, 16)
                z = (accv[pl.ds(p * CHUNK + j * 128 + k * 16, 16)] + g1[r, sl]
                     + g2[r, sl] + g3[r, sl] + g4[r, sl] + g5[r, sl])
                z = jnp.minimum(jnp.maximum(z, -30.0), 30.0)
                outv[pl.ds(p * CHUNK + j * 128 + k * 16, 16)] = (
                    1.0 / (1.0 + jnp.exp(-z)))
            return 0

        lax.fori_loop(0, NROW, out_body, 0)
        pltpu.sync_copy(outv.at[pl.ds(p * CHUNK, CHUNK)],
                        out.at[pl.ds(base_e, CHUNK)])

    base0 = wid * PER_W
    extract(0, base0)
    fire(0)

    def chunk_body(ck, _):
        p = lax.rem(ck, 2)
        q = 1 - p
        base_e = base0 + ck * CHUNK
        extract(p, base_e)
        drain(q)
        fire(p)
        combine(q, base_e - CHUNK)
        return 0

    lax.fori_loop(1, NCHUNK, chunk_body, 0)
    pl_last = (NCHUNK - 1) % 2
    drain(pl_last)
    combine(pl_last, base0 + (NCHUNK - 1) * CHUNK)


def _sc_gather(xf, svals, wb):
    mesh = plsc.VectorSubcoreMesh(core_axis_name="c", subcore_axis_name="s")
    f = functools.partial(
        pl.kernel, _sc_body, mesh=mesh,
        out_type=jax.ShapeDtypeStruct((E,), jnp.float32),
        scratch_types=[
            pltpu.VMEM((2 * CHUNK * NUM_FEAT,), jnp.float32),
            pltpu.VMEM((2 * CHUNK,), jnp.float32),
        ] + [pltpu.VMEM((2 * NROW, 128), jnp.int32)] * 5
          + [pltpu.VMEM((2 * NROW, 128), jnp.float32)] * 5
          + [pltpu.VMEM((2 * CHUNK,), jnp.float32),
             pltpu.VMEM((16,), jnp.float32),
             pltpu.SemaphoreType.DMA],
        compiler_params=pltpu.CompilerParams(needs_layout_passes=False),
    )()
    return f(xf, *svals, wb)


def kernel(x, emb_1, emb_3, emb_4, emb_6, emb_7, W, b):
    svals = _scalarize((emb_1, emb_3, emb_4, emb_6, emb_7), W)
    wnum = jnp.stack([W[o, 0] for o in NUM_OFF])
    wnum = wnum.astype(jnp.bfloat16).astype(jnp.float32)
    wb = jnp.concatenate([wnum, b, jnp.zeros((6,), jnp.float32)])
    xf = x.reshape(-1)
    o = _sc_gather(xf, svals, wb)
    return o.reshape(B, L)
